# baseline (speedup 1.0000x reference)
import jax
import jax.numpy as jnp
from jax import lax
from jax.experimental import pallas as pl
from jax.experimental.pallas import tpu as pltpu
from jax.experimental.pallas import tpu_sc as plsc

_E = 64
_K = 8
_D = 4096
_TOKENS = 16384
_NW = 32
_CHUNKS = (7168, 7168, 2048)
_R = 512


_RB = 256
_NBUF = 8


def _make_router_tc(row0, rows):
    nblk = rows // _RB

    def body(hs_hbm, wt_ref, probs_ref, *rest):
        bufs = rest[:_NBUF]
        sem = rest[_NBUF]

        def start(i, b):
            pltpu.make_async_copy(
                hs_hbm.at[pl.ds(row0 + i * _RB, _RB), :],
                bufs[b], sem.at[b]).start()

        for b in range(min(_NBUF, nblk)):
            start(b, b)
        for i in range(nblk):
            b = i % _NBUF
            pltpu.make_async_copy(
                hs_hbm.at[pl.ds(row0 + i * _RB, _RB), :],
                bufs[b], sem.at[b]).wait()
            logits = jnp.dot(bufs[b][...], wt_ref[...],
                             preferred_element_type=jnp.float32)
            m = jnp.max(logits, axis=-1, keepdims=True)
            ex = jnp.exp(logits - m)
            probs_ref[pl.ds(i * _RB, _RB), :] = (
                ex / jnp.sum(ex, axis=-1, keepdims=True))
            nxt = i + _NBUF
            if nxt < nblk:
                start(nxt, b)

    return pl.pallas_call(
        body,
        in_specs=[
            pl.BlockSpec(memory_space=pl.ANY),
            pl.BlockSpec((_D, _E), lambda: (0, 0)),
        ],
        out_specs=pl.BlockSpec((rows, _E), lambda: (0, 0)),
        out_shape=jax.ShapeDtypeStruct((rows, _E), jnp.float32),
        scratch_shapes=(
            [pltpu.VMEM((_RB, _D), jnp.float32)] * _NBUF
            + [pltpu.SemaphoreType.DMA((_NBUF,))]
        ),
    )


def _make_sc_topk(rows):
    rpw = rows // _NW
    groups = rpw // 16

    def _sc_topk_body(probs_hbm, scores_hbm, idx_hbm, probs_v, scores_v, idx_v):
        wid = lax.axis_index("s") * 2 + lax.axis_index("c")
        base = wid * rpw
        pltpu.sync_copy(probs_hbm.at[pl.ds(base * _E, rpw * _E)], probs_v)

        lane = lax.broadcasted_iota(jnp.int32, (16,), 0)
        lane_probs = lane * _E
        lane_out = lane * _K

        def group(g, carry):
            idx0 = lane_probs + g * (16 * _E)
            vals = [jnp.full((16,), -1.0, jnp.float32)] * _K
            idxs = [jnp.zeros((16,), jnp.int32)] * _K
            for e in range(_E):
                v = plsc.load_gather(probs_v, (idx0 + e,))
                ei = jnp.full((16,), e, jnp.int32)
                c = [v > vals[j] for j in range(_K)]
                new_vals = [jnp.where(c[0], v, vals[0])]
                new_idxs = [jnp.where(c[0], ei, idxs[0])]
                for j in range(1, _K):
                    new_vals.append(jnp.where(c[j - 1], vals[j - 1],
                                              jnp.where(c[j], v, vals[j])))
                    new_idxs.append(jnp.where(c[j - 1], idxs[j - 1],
                                              jnp.where(c[j], ei, idxs[j])))
                vals, idxs = new_vals, new_idxs
            s = vals[0]
            for j in range(1, _K):
                s = s + vals[j]
            r = 1.0 / s
            obase = lane_out + g * (16 * _K)
            for j in range(_K):
                plsc.store_scatter(scores_v, (obase + j,), vals[j] * r)
                plsc.store_scatter(idx_v, (obase + j,), idxs[j])
            return carry

        lax.fori_loop(0, groups, group, 0)
        pltpu.sync_copy(scores_v, scores_hbm.at[pl.ds(base * _K, rpw * _K)])
        pltpu.sync_copy(idx_v, idx_hbm.at[pl.ds(base * _K, rpw * _K)])

    return pl.kernel(
        _sc_topk_body,
        out_type=[
            jax.ShapeDtypeStruct((rows * _K,), jnp.float32),
            jax.ShapeDtypeStruct((rows * _K,), jnp.int32),
        ],
        mesh=plsc.VectorSubcoreMesh(core_axis_name="c", subcore_axis_name="s"),
        scratch_types=[
            pltpu.VMEM((rpw * _E,), jnp.float32),
            pltpu.VMEM((rpw * _K,), jnp.float32),
            pltpu.VMEM((rpw * _K,), jnp.int32),
        ],
        compiler_params=pltpu.CompilerParams(needs_layout_passes=False),
    )


_row0s = [sum(_CHUNKS[:c]) for c in range(len(_CHUNKS))]
_router_tc = [_make_router_tc(_row0s[c], _CHUNKS[c])
              for c in range(len(_CHUNKS))]
_sc_topk = {rows: _make_sc_topk(rows) for rows in set(_CHUNKS)}


def kernel(hidden_states, weight):
    hs = hidden_states.reshape(-1, _D)
    wt = weight.T
    probs_chunks = []
    score_chunks = []
    idx_chunks = []
    for c, rows in enumerate(_CHUNKS):
        probs_c = _router_tc[c](hs, wt)
        s_c, i_c = _sc_topk[rows](probs_c.reshape(-1))
        probs_chunks.append(probs_c)
        score_chunks.append(s_c.reshape(rows, _K))
        idx_chunks.append(i_c.reshape(rows, _K))
    router_logits = jnp.concatenate(probs_chunks, axis=0)
    router_scores = jnp.concatenate(score_chunks, axis=0)
    router_indices = jnp.concatenate(idx_chunks, axis=0)
    return (router_logits, router_scores, router_indices)


# speedup vs baseline: 1.1063x; 1.1063x over previous
import jax
import jax.numpy as jnp
from jax import lax
from jax.experimental import pallas as pl
from jax.experimental.pallas import tpu as pltpu
from jax.experimental.pallas import tpu_sc as plsc

_E = 64
_K = 8
_D = 4096
_TOKENS = 16384
_NW = 32
_CHUNKS = (7168, 7168, 2048)
_R = 512


def _router_tc_body(hs_ref, wt_ref, out_ref):
    logits = jnp.dot(hs_ref[...], wt_ref[...],
                     preferred_element_type=jnp.float32)
    m = jnp.max(logits, axis=-1, keepdims=True)
    ex = jnp.exp(logits - m)
    out_ref[...] = ex / jnp.sum(ex, axis=-1, keepdims=True)


def _make_router_tc(row0, rows):
    base = row0 // _R
    return pl.pallas_call(
        _router_tc_body,
        grid=(rows // _R,),
        in_specs=[
            pl.BlockSpec((_R, _D), lambda i: (base + i, 0)),
            pl.BlockSpec((_D, _E), lambda i: (0, 0)),
        ],
        out_specs=pl.BlockSpec((_R, _E), lambda i: (i, 0)),
        out_shape=jax.ShapeDtypeStruct((rows, _E), jnp.float32),
    )


def _make_sc_topk(rows):
    rpw = rows // _NW
    groups = rpw // 16

    def _sc_topk_body(probs_hbm, scores_hbm, idx_hbm, probs_v, scores_v, idx_v):
        wid = lax.axis_index("s") * 2 + lax.axis_index("c")
        base = wid * rpw
        pltpu.sync_copy(probs_hbm.at[pl.ds(base * _E, rpw * _E)], probs_v)

        lane = lax.broadcasted_iota(jnp.int32, (16,), 0)
        lane_probs = lane * _E
        lane_out = lane * _K

        def group(g, carry):
            idx0 = lane_probs + g * (16 * _E)
            vals = [jnp.full((16,), -1.0, jnp.float32)] * _K
            idxs = [jnp.zeros((16,), jnp.int32)] * _K
            for e in range(_E):
                v = plsc.load_gather(probs_v, (idx0 + e,))
                ei = jnp.full((16,), e, jnp.int32)
                c = [v > vals[j] for j in range(_K)]
                new_vals = [jnp.where(c[0], v, vals[0])]
                new_idxs = [jnp.where(c[0], ei, idxs[0])]
                for j in range(1, _K):
                    new_vals.append(jnp.where(c[j - 1], vals[j - 1],
                                              jnp.where(c[j], v, vals[j])))
                    new_idxs.append(jnp.where(c[j - 1], idxs[j - 1],
                                              jnp.where(c[j], ei, idxs[j])))
                vals, idxs = new_vals, new_idxs
            s = vals[0]
            for j in range(1, _K):
                s = s + vals[j]
            r = 1.0 / s
            obase = lane_out + g * (16 * _K)
            for j in range(_K):
                plsc.store_scatter(scores_v, (obase + j,), vals[j] * r)
                plsc.store_scatter(idx_v, (obase + j,), idxs[j])
            return carry

        lax.fori_loop(0, groups, group, 0)
        pltpu.sync_copy(scores_v, scores_hbm.at[pl.ds(base * _K, rpw * _K)])
        pltpu.sync_copy(idx_v, idx_hbm.at[pl.ds(base * _K, rpw * _K)])

    return pl.kernel(
        _sc_topk_body,
        out_type=[
            jax.ShapeDtypeStruct((rows * _K,), jnp.float32),
            jax.ShapeDtypeStruct((rows * _K,), jnp.int32),
        ],
        mesh=plsc.VectorSubcoreMesh(core_axis_name="c", subcore_axis_name="s"),
        scratch_types=[
            pltpu.VMEM((rpw * _E,), jnp.float32),
            pltpu.VMEM((rpw * _K,), jnp.float32),
            pltpu.VMEM((rpw * _K,), jnp.int32),
        ],
        compiler_params=pltpu.CompilerParams(needs_layout_passes=False),
    )


_row0s = [sum(_CHUNKS[:c]) for c in range(len(_CHUNKS))]
_router_tc = [_make_router_tc(_row0s[c], _CHUNKS[c])
              for c in range(len(_CHUNKS))]
_sc_topk = {rows: _make_sc_topk(rows) for rows in set(_CHUNKS)}


def kernel(hidden_states, weight):
    hs = hidden_states.reshape(-1, _D)
    wt = weight.T
    probs_chunks = []
    score_chunks = []
    idx_chunks = []
    for c, rows in enumerate(_CHUNKS):
        probs_c = _router_tc[c](hs, wt)
        s_c, i_c = _sc_topk[rows](probs_c.reshape(-1))
        probs_chunks.append(probs_c)
        score_chunks.append(s_c.reshape(rows, _K))
        idx_chunks.append(i_c.reshape(rows, _K))
    router_logits = jnp.concatenate(probs_chunks, axis=0)
    router_scores = jnp.concatenate(score_chunks, axis=0)
    router_indices = jnp.concatenate(idx_chunks, axis=0)
    return (router_logits, router_scores, router_indices)
